# baseline (speedup 1.0000x reference)
import jax
import jax.numpy as jnp
from jax.experimental import pallas as pl
from jax.experimental.pallas import tpu as pltpu

N, C, H, O = 512, 64, 128, 64
NBITS = 7
TT = 128
TJ = 8
NSTEPS = N // TJ

_NAMES = ('op0', 'red0', 'exp1', 'op1', 'red1', 'exp2', 'op2')


def _transpose_body(src, dst):
    dst[...] = jnp.swapaxes(src[...], 0, 1)


def _transpose(x2s):
    return pl.pallas_call(
        _transpose_body,
        grid=(N // TT, N // TT),
        in_specs=[pl.BlockSpec((TT, TT, C), lambda i, j: (j, i, 0))],
        out_specs=pl.BlockSpec((TT, TT, C), lambda i, j: (i, j, 0)),
        out_shape=jax.ShapeDtypeStruct((N, N, C), jnp.float32),
        compiler_params=pltpu.CompilerParams(
            dimension_semantics=("arbitrary", "arbitrary"),
        ),
    )(x2s)


def _body(*refs):
    (gates, x0, x1, rows, trows), rest = refs[:5], refs[5:]
    w = dict(zip(
        [n + s for n in _NAMES for s in ('_W1', '_b1', '_W2', '_b2')],
        rest[:28]))
    out0, out1, out2, af, bfac, mx, mn = rest[28:]

    jb = pl.program_id(0)
    f32 = jnp.float32

    def g(k):
        return gates[k:k + 1, :O]

    def mlp(x, name):
        h = jnp.maximum(
            jnp.dot(x, w[name + '_W1'][...], preferred_element_type=f32)
            + w[name + '_b1'][...], 0.0)
        return (jnp.dot(h, w[name + '_W2'][...], preferred_element_type=f32)
                + w[name + '_b2'][...])

    @pl.when(jb == 0)
    def _():
        x1f = x1[...]
        af[...] = jnp.dot(x1f, w['exp2_W1'][0:C, :], preferred_element_type=f32)
        bfac[...] = jnp.dot(x1f, w['exp2_W1'][C:2 * C, :], preferred_element_type=f32)
        mx[...] = jnp.zeros((N, C), f32)
        mn[...] = jnp.ones((N, C), f32)
        r1 = jnp.concatenate([jnp.max(x1f, axis=0, keepdims=True),
                              jnp.min(x1f, axis=0, keepdims=True)], axis=-1)
        s0 = mlp(x0[...], 'op0') * g(0) + mlp(r1, 'red0') * g(1)
        out0[...] = jax.nn.sigmoid(s0) * g(7)

    rows_flat = rows[...].reshape(TJ * N, C)
    trows_val = trows[...]
    rm = jnp.dot(rows_flat, w['op2_W1'][0:C, :], preferred_element_type=f32)
    cm = jnp.dot(trows_val.reshape(TJ * N, C), w['op2_W1'][C:2 * C, :],
                 preferred_element_type=f32)
    h2 = jnp.maximum(rm + cm + w['op2_b1'][...], 0.0)
    a_j = af[pl.ds(jb * TJ, TJ), :]
    he = jnp.maximum(
        (a_j[:, None, :] + bfac[...][None, :, :]).reshape(TJ * N, H)
        + w['exp2_b1'][...], 0.0)
    s2 = ((jnp.dot(h2, w['op2_W2'][...], preferred_element_type=f32)
           + w['op2_b2'][...]) * g(6)
          + (jnp.dot(he, w['exp2_W2'][...], preferred_element_type=f32)
             + w['exp2_b2'][...]) * g(5))
    out2[...] = (jax.nn.sigmoid(s2) * g(9)).reshape(TJ, N, O)

    rid = jax.lax.broadcasted_iota(jnp.int32, (N, C), 0)
    mxv, mnv = mx[...], mn[...]
    for k in range(TJ):
        col_k = trows_val[k]
        dmask = rid == (jb * TJ + k)
        mxv = jnp.maximum(mxv, jnp.where(dmask, 0.0, col_k))
        mnv = jnp.minimum(mnv, jnp.where(dmask, 1.0, col_k))
    mx[...] = mxv
    mn[...] = mnv

    @pl.when(jb == NSTEPS - 1)
    def _():
        red = jnp.concatenate([mx[...], mn[...]], axis=-1)
        s1 = (mlp(red, 'red1') * g(4) + mlp(x1[...], 'op1') * g(3)
              + mlp(x0[...], 'exp1') * g(2))
        out1[...] = jax.nn.sigmoid(s1) * g(8)


def kernel(x0, x1, x2, params, action):
    f32 = jnp.float32
    x1s = x1.reshape(N, C)
    x2s = x2.reshape(N, N, C)
    x2t = _transpose(x2s)

    a = jnp.asarray(action, jnp.int32)
    bfs = [((a >> (NBITS - 1 - k)) & 1).astype(f32) for k in range(NBITS)]
    act0 = (bfs[0] + bfs[1] > 0).astype(f32)
    act1 = (bfs[2] + bfs[3] + bfs[4] > 0).astype(f32)
    act2 = (bfs[5] + bfs[6] > 0).astype(f32)
    gvec = jnp.stack(bfs + [act0, act1, act2] + [jnp.zeros(())] * 6)
    gates = jnp.broadcast_to(gvec[:, None], (16, 128)).astype(f32)

    weights = []
    wspecs = []
    for name in _NAMES:
        for suff in ('_W1', '_b1', '_W2', '_b2'):
            wgt = params[name + suff]
            if wgt.ndim == 1:
                wgt = wgt.reshape(1, -1)
            weights.append(wgt)
            wspecs.append(pl.BlockSpec(wgt.shape, lambda jb: (0, 0)))

    out0, out1, out2 = pl.pallas_call(
        _body,
        grid=(NSTEPS,),
        in_specs=[
            pl.BlockSpec((16, 128), lambda jb: (0, 0)),
            pl.BlockSpec((1, C), lambda jb: (0, 0)),
            pl.BlockSpec((N, C), lambda jb: (0, 0)),
            pl.BlockSpec((TJ, N, C), lambda jb: (jb, 0, 0)),
            pl.BlockSpec((TJ, N, C), lambda jb: (jb, 0, 0)),
        ] + wspecs,
        out_specs=[
            pl.BlockSpec((1, O), lambda jb: (0, 0)),
            pl.BlockSpec((N, O), lambda jb: (0, 0)),
            pl.BlockSpec((TJ, N, O), lambda jb: (jb, 0, 0)),
        ],
        out_shape=[
            jax.ShapeDtypeStruct((1, O), f32),
            jax.ShapeDtypeStruct((N, O), f32),
            jax.ShapeDtypeStruct((N, N, O), f32),
        ],
        scratch_shapes=[
            pltpu.VMEM((N, H), f32),
            pltpu.VMEM((N, H), f32),
            pltpu.VMEM((N, C), f32),
            pltpu.VMEM((N, C), f32),
        ],
        compiler_params=pltpu.CompilerParams(
            dimension_semantics=("arbitrary",),
        ),
    )(gates, x0, x1s, x2s, x2t, *weights)

    return out0, out1.reshape(1, N, O), out2.reshape(1, N, N, O)


# speedup vs baseline: 1.1353x; 1.1353x over previous
import jax
import jax.numpy as jnp
from jax.experimental import pallas as pl
from jax.experimental.pallas import tpu as pltpu

N, C, H, O = 512, 64, 128, 64
NBITS = 7
TT = 128
TI = 8
NSTEPS = N // TI

_NAMES = ('op0', 'red0', 'exp1', 'op1', 'red1', 'exp2', 'op2')


def _transpose_body(src, dst):
    dst[...] = jnp.transpose(src[...], (2, 1, 0))


def _transpose(x2m):
    return pl.pallas_call(
        _transpose_body,
        grid=(N // TT, N // TT),
        in_specs=[pl.BlockSpec((TT, C, TT), lambda p, q: (q, 0, p))],
        out_specs=pl.BlockSpec((TT, C, TT), lambda p, q: (p, 0, q)),
        out_shape=jax.ShapeDtypeStruct((N, C, N), jnp.float32),
        compiler_params=pltpu.CompilerParams(
            dimension_semantics=("arbitrary", "arbitrary"),
        ),
    )(x2m)


def _body(*refs):
    (gates, x0m, x1m, rows, trows), rest = refs[:5], refs[5:]
    w = dict(zip(
        [n + s for n in _NAMES for s in ('_W1T', '_b1', '_W2T', '_b2')],
        rest[:28]))
    out0, out1, out2, af, bfac, mx, mn = rest[28:]

    ib = pl.program_id(0)
    f32 = jnp.float32

    def g(k):
        return gates[k:k + 1, 0:1]

    def mlp(x, name):
        h = jnp.maximum(
            jnp.dot(w[name + '_W1T'][...], x, preferred_element_type=f32)
            + w[name + '_b1'][...], 0.0)
        return (jnp.dot(w[name + '_W2T'][...], h, preferred_element_type=f32)
                + w[name + '_b2'][...])

    @pl.when(ib == 0)
    def _():
        x1v = x1m[...]
        af[...] = jnp.dot(w['exp2_W1T'][:, 0:C], x1v, preferred_element_type=f32)
        bfac[...] = (jnp.dot(w['exp2_W1T'][:, C:2 * C], x1v,
                             preferred_element_type=f32) + w['exp2_b1'][...])
        mx[...] = jnp.zeros((C, N), f32)
        mn[...] = jnp.ones((C, N), f32)
        r1 = jnp.concatenate([jnp.max(x1v, axis=1, keepdims=True),
                              jnp.min(x1v, axis=1, keepdims=True)], axis=0)
        s0 = mlp(x0m[...], 'op0') * g(0) + mlp(r1, 'red0') * g(1)
        out0[...] = jax.nn.sigmoid(s0) * g(7)

    rows_val = rows[...]
    trows_val = trows[...]
    li = jax.lax.broadcasted_iota(jnp.int32, (C, N), 1)
    ri = jax.lax.broadcasted_iota(jnp.int32, (N, 1), 0)
    mxv, mnv = mx[...], mn[...]
    w1t = w['op2_W1T'][...]
    af_val = af[...]
    for k in range(TI):
        ik = ib * TI + k
        rm_k = jnp.dot(w1t[:, 0:C], rows_val[k], preferred_element_type=f32)
        cm_k = jnp.dot(w1t[:, C:2 * C], trows_val[k], preferred_element_type=f32)
        h2_k = jnp.maximum(rm_k + cm_k + w['op2_b1'][...], 0.0)
        onehot = (ri == ik).astype(f32)
        a_col = jnp.dot(af_val, onehot, preferred_element_type=f32)
        he_k = jnp.maximum(a_col + bfac[...], 0.0)
        s2_k = ((jnp.dot(w['op2_W2T'][...], h2_k, preferred_element_type=f32)
                 + w['op2_b2'][...]) * g(6)
                + (jnp.dot(w['exp2_W2T'][...], he_k, preferred_element_type=f32)
                   + w['exp2_b2'][...]) * g(5))
        out2[k, :, :] = jax.nn.sigmoid(s2_k) * g(9)
        dmask = li == ik
        mxv = jnp.maximum(mxv, jnp.where(dmask, 0.0, trows_val[k]))
        mnv = jnp.minimum(mnv, jnp.where(dmask, 1.0, trows_val[k]))
    mx[...] = mxv
    mn[...] = mnv

    @pl.when(ib == NSTEPS - 1)
    def _():
        red = jnp.concatenate([mx[...], mn[...]], axis=0)
        s1 = (mlp(red, 'red1') * g(4) + mlp(x1m[...], 'op1') * g(3)
              + mlp(x0m[...], 'exp1') * g(2))
        out1[...] = jax.nn.sigmoid(s1) * g(8)


def kernel(x0, x1, x2, params, action):
    f32 = jnp.float32
    x0m = jnp.transpose(x0, (1, 0))
    x1m = jnp.transpose(x1, (0, 2, 1)).reshape(C, N)
    x2m = jnp.transpose(x2, (0, 1, 3, 2)).reshape(N, C, N)
    x2tm = _transpose(x2m)

    a = jnp.asarray(action, jnp.int32)
    bfs = [((a >> (NBITS - 1 - k)) & 1).astype(f32) for k in range(NBITS)]
    act0 = (bfs[0] + bfs[1] > 0).astype(f32)
    act1 = (bfs[2] + bfs[3] + bfs[4] > 0).astype(f32)
    act2 = (bfs[5] + bfs[6] > 0).astype(f32)
    gvec = jnp.stack(bfs + [act0, act1, act2] + [jnp.zeros(())] * 6)
    gates = jnp.broadcast_to(gvec[:, None], (16, 128)).astype(f32)

    weights = []
    wspecs = []
    for name in _NAMES:
        for suff in ('_W1', '_b1', '_W2', '_b2'):
            wgt = params[name + suff]
            if wgt.ndim == 1:
                wgt = wgt.reshape(-1, 1)
            else:
                wgt = wgt.T
            weights.append(wgt)
            wspecs.append(pl.BlockSpec(wgt.shape, lambda ib: (0, 0)))

    out0m, out1m, out2m = pl.pallas_call(
        _body,
        grid=(NSTEPS,),
        in_specs=[
            pl.BlockSpec((16, 128), lambda ib: (0, 0)),
            pl.BlockSpec((C, 1), lambda ib: (0, 0)),
            pl.BlockSpec((C, N), lambda ib: (0, 0)),
            pl.BlockSpec((TI, C, N), lambda ib: (ib, 0, 0)),
            pl.BlockSpec((TI, C, N), lambda ib: (ib, 0, 0)),
        ] + wspecs,
        out_specs=[
            pl.BlockSpec((O, 1), lambda ib: (0, 0)),
            pl.BlockSpec((O, N), lambda ib: (0, 0)),
            pl.BlockSpec((TI, O, N), lambda ib: (ib, 0, 0)),
        ],
        out_shape=[
            jax.ShapeDtypeStruct((O, 1), f32),
            jax.ShapeDtypeStruct((O, N), f32),
            jax.ShapeDtypeStruct((N, O, N), f32),
        ],
        scratch_shapes=[
            pltpu.VMEM((H, N), f32),
            pltpu.VMEM((H, N), f32),
            pltpu.VMEM((C, N), f32),
            pltpu.VMEM((C, N), f32),
        ],
        compiler_params=pltpu.CompilerParams(
            dimension_semantics=("arbitrary",),
        ),
    )(gates, x0m, x1m, x2m, x2tm, *weights)

    out0 = jnp.transpose(out0m, (1, 0))
    out1 = jnp.transpose(out1m[None], (0, 2, 1))
    out2 = jnp.transpose(out2m[None], (0, 1, 3, 2))
    return out0, out1, out2


# speedup vs baseline: 1.1438x; 1.0075x over previous
import jax
import jax.numpy as jnp
from jax.experimental import pallas as pl
from jax.experimental.pallas import tpu as pltpu

N, C, H, O = 512, 64, 128, 64
NBITS = 7
TT = 128
TI = 16
NSTEPS = N // TI

_NAMES = ('op0', 'red0', 'exp1', 'op1', 'red1', 'exp2', 'op2')


def _transpose_body(src, dst):
    dst[...] = jnp.transpose(src[...], (2, 1, 0))


def _transpose(x2m):
    return pl.pallas_call(
        _transpose_body,
        grid=(N // TT, N // TT),
        in_specs=[pl.BlockSpec((TT, C, TT), lambda p, q: (q, 0, p))],
        out_specs=pl.BlockSpec((TT, C, TT), lambda p, q: (p, 0, q)),
        out_shape=jax.ShapeDtypeStruct((N, C, N), jnp.float32),
        compiler_params=pltpu.CompilerParams(
            dimension_semantics=("arbitrary", "arbitrary"),
        ),
    )(x2m)


def _body(*refs):
    (gates, x0m, x1m, rows, trows), rest = refs[:5], refs[5:]
    w = dict(zip(
        [n + s for n in _NAMES for s in ('_W1T', '_b1', '_W2T', '_b2')],
        rest[:28]))
    out0, out1, out2, af, bfac, mx, mn = rest[28:]

    ib = pl.program_id(0)
    f32 = jnp.float32

    def g(k):
        return gates[k:k + 1, 0:1]

    def mlp(x, name):
        h = jnp.maximum(
            jnp.dot(w[name + '_W1T'][...], x, preferred_element_type=f32)
            + w[name + '_b1'][...], 0.0)
        return (jnp.dot(w[name + '_W2T'][...], h, preferred_element_type=f32)
                + w[name + '_b2'][...])

    @pl.when(ib == 0)
    def _():
        x1v = x1m[...]
        af[...] = jnp.dot(w['exp2_W1T'][:, 0:C], x1v, preferred_element_type=f32)
        bfac[...] = (jnp.dot(w['exp2_W1T'][:, C:2 * C], x1v,
                             preferred_element_type=f32) + w['exp2_b1'][...])
        mx[...] = jnp.zeros((C, N), f32)
        mn[...] = jnp.ones((C, N), f32)
        r1 = jnp.concatenate([jnp.max(x1v, axis=1, keepdims=True),
                              jnp.min(x1v, axis=1, keepdims=True)], axis=0)
        s0 = mlp(x0m[...], 'op0') * g(0) + mlp(r1, 'red0') * g(1)
        out0[...] = jax.nn.sigmoid(s0) * g(7)

    rows_val = rows[...]
    trows_val = trows[...]
    ri = jax.lax.broadcasted_iota(jnp.int32, (N, 1), 0)
    w1t = w['op2_W1T'][...]
    af_val = af[...]
    bf_val = bfac[...]
    b1o = w['op2_b1'][...]
    w2o_s = w['op2_W2T'][...] * g(6)
    w2e_s = w['exp2_W2T'][...] * g(5)
    b2_s = w['op2_b2'][...] * g(6) + w['exp2_b2'][...] * g(5)
    act2 = g(9)
    for k in range(TI):
        ik = ib * TI + k
        rm_k = jnp.dot(w1t[:, 0:C], rows_val[k], preferred_element_type=f32)
        cm_k = jnp.dot(w1t[:, C:2 * C], trows_val[k], preferred_element_type=f32)
        h2_k = jnp.maximum(rm_k + cm_k + b1o, 0.0)
        onehot = (ri == ik).astype(f32)
        a_col = jnp.dot(af_val, onehot, preferred_element_type=f32)
        he_k = jnp.maximum(a_col + bf_val, 0.0)
        s2_k = (jnp.dot(w2o_s, h2_k, preferred_element_type=f32)
                + jnp.dot(w2e_s, he_k, preferred_element_type=f32) + b2_s)
        out2[k, :, :] = jax.nn.sigmoid(s2_k) * act2
    ki3 = jax.lax.broadcasted_iota(jnp.int32, (TI, C, N), 0)
    ji3 = jax.lax.broadcasted_iota(jnp.int32, (TI, C, N), 2)
    dmask3 = ji3 == (ki3 + ib * TI)
    mx[...] = jnp.maximum(mx[...], jnp.max(jnp.where(dmask3, 0.0, trows_val), axis=0))
    mn[...] = jnp.minimum(mn[...], jnp.min(jnp.where(dmask3, 1.0, trows_val), axis=0))

    @pl.when(ib == NSTEPS - 1)
    def _():
        red = jnp.concatenate([mx[...], mn[...]], axis=0)
        s1 = (mlp(red, 'red1') * g(4) + mlp(x1m[...], 'op1') * g(3)
              + mlp(x0m[...], 'exp1') * g(2))
        out1[...] = jax.nn.sigmoid(s1) * g(8)


def kernel(x0, x1, x2, params, action):
    f32 = jnp.float32
    x0m = jnp.transpose(x0, (1, 0))
    x1m = jnp.transpose(x1, (0, 2, 1)).reshape(C, N)
    x2m = jnp.transpose(x2, (0, 1, 3, 2)).reshape(N, C, N)
    x2tm = _transpose(x2m)

    a = jnp.asarray(action, jnp.int32)
    bfs = [((a >> (NBITS - 1 - k)) & 1).astype(f32) for k in range(NBITS)]
    act0 = (bfs[0] + bfs[1] > 0).astype(f32)
    act1 = (bfs[2] + bfs[3] + bfs[4] > 0).astype(f32)
    act2 = (bfs[5] + bfs[6] > 0).astype(f32)
    gvec = jnp.stack(bfs + [act0, act1, act2] + [jnp.zeros(())] * 6)
    gates = jnp.broadcast_to(gvec[:, None], (16, 128)).astype(f32)

    weights = []
    wspecs = []
    for name in _NAMES:
        for suff in ('_W1', '_b1', '_W2', '_b2'):
            wgt = params[name + suff]
            if wgt.ndim == 1:
                wgt = wgt.reshape(-1, 1)
            else:
                wgt = wgt.T
            weights.append(wgt)
            wspecs.append(pl.BlockSpec(wgt.shape, lambda ib: (0, 0)))

    out0m, out1m, out2m = pl.pallas_call(
        _body,
        grid=(NSTEPS,),
        in_specs=[
            pl.BlockSpec((16, 128), lambda ib: (0, 0)),
            pl.BlockSpec((C, 1), lambda ib: (0, 0)),
            pl.BlockSpec((C, N), lambda ib: (0, 0)),
            pl.BlockSpec((TI, C, N), lambda ib: (ib, 0, 0)),
            pl.BlockSpec((TI, C, N), lambda ib: (ib, 0, 0)),
        ] + wspecs,
        out_specs=[
            pl.BlockSpec((O, 1), lambda ib: (0, 0)),
            pl.BlockSpec((O, N), lambda ib: (0, 0)),
            pl.BlockSpec((TI, O, N), lambda ib: (ib, 0, 0)),
        ],
        out_shape=[
            jax.ShapeDtypeStruct((O, 1), f32),
            jax.ShapeDtypeStruct((O, N), f32),
            jax.ShapeDtypeStruct((N, O, N), f32),
        ],
        scratch_shapes=[
            pltpu.VMEM((H, N), f32),
            pltpu.VMEM((H, N), f32),
            pltpu.VMEM((C, N), f32),
            pltpu.VMEM((C, N), f32),
        ],
        compiler_params=pltpu.CompilerParams(
            dimension_semantics=("arbitrary",),
        ),
    )(gates, x0m, x1m, x2m, x2tm, *weights)

    out0 = jnp.transpose(out0m, (1, 0))
    out1 = jnp.transpose(out1m[None], (0, 2, 1))
    out2 = jnp.transpose(out2m[None], (0, 1, 3, 2))
    return out0, out1, out2


# speedup vs baseline: 1.2307x; 1.0759x over previous
import jax
import jax.numpy as jnp
from jax.experimental import pallas as pl
from jax.experimental.pallas import tpu as pltpu

N, C, H, O = 512, 64, 128, 64
NBITS = 7
TT = 128
TI = 16
NSTEPS = N // TI

_NAMES = ('op0', 'red0', 'exp1', 'op1', 'red1', 'exp2', 'op2')


def _transpose_body(src, dst_n, dst_t):
    srcb = src[...].astype(jnp.bfloat16)
    dst_n[...] = srcb
    dst_t[...] = jnp.transpose(srcb, (2, 1, 0))


def _transpose(x2m):
    return pl.pallas_call(
        _transpose_body,
        grid=(N // TT, N // TT),
        in_specs=[pl.BlockSpec((TT, C, TT), lambda p, q: (q, 0, p))],
        out_specs=[
            pl.BlockSpec((TT, C, TT), lambda p, q: (q, 0, p)),
            pl.BlockSpec((TT, C, TT), lambda p, q: (p, 0, q)),
        ],
        out_shape=[
            jax.ShapeDtypeStruct((N, C, N), jnp.bfloat16),
            jax.ShapeDtypeStruct((N, C, N), jnp.bfloat16),
        ],
        compiler_params=pltpu.CompilerParams(
            dimension_semantics=("arbitrary", "arbitrary"),
        ),
    )(x2m)


def _body(*refs):
    (gates, x0m, x1m, rows, trows), rest = refs[:5], refs[5:]
    w = dict(zip(
        [n + s for n in _NAMES for s in ('_W1T', '_b1', '_W2T', '_b2')],
        rest[:28]))
    out0, out1, out2, af, bfac, mx, mn = rest[28:]

    ib = pl.program_id(0)
    f32 = jnp.float32

    def g(k):
        return gates[k:k + 1, 0:1]

    def mlp(x, name):
        h = jnp.maximum(
            jnp.dot(w[name + '_W1T'][...], x, preferred_element_type=f32)
            + w[name + '_b1'][...], 0.0)
        return (jnp.dot(w[name + '_W2T'][...], h, preferred_element_type=f32)
                + w[name + '_b2'][...])

    @pl.when(ib == 0)
    def _():
        x1v = x1m[...]
        af[...] = jnp.dot(w['exp2_W1T'][:, 0:C], x1v, preferred_element_type=f32)
        bfac[...] = (jnp.dot(w['exp2_W1T'][:, C:2 * C], x1v,
                             preferred_element_type=f32) + w['exp2_b1'][...])
        mx[...] = jnp.zeros((C, N), f32)
        mn[...] = jnp.ones((C, N), f32)
        r1 = jnp.concatenate([jnp.max(x1v, axis=1, keepdims=True),
                              jnp.min(x1v, axis=1, keepdims=True)], axis=0)
        s0 = mlp(x0m[...], 'op0') * g(0) + mlp(r1, 'red0') * g(1)
        out0[...] = jax.nn.sigmoid(s0) * g(7)

    rows_val = rows[...]
    trows_val = trows[...]
    bf16 = jnp.bfloat16
    ri = jax.lax.broadcasted_iota(jnp.int32, (N, 1), 0)
    w1t = w['op2_W1T'][...].astype(bf16)
    af_val = af[...]
    bf_val = bfac[...]
    b1o = w['op2_b1'][...]
    w2o_s = (w['op2_W2T'][...] * g(6)).astype(bf16)
    w2e_s = (w['exp2_W2T'][...] * g(5)).astype(bf16)
    b2_s = w['op2_b2'][...] * g(6) + w['exp2_b2'][...] * g(5)
    act2 = g(9)
    for k in range(TI):
        ik = ib * TI + k
        rm_k = jnp.dot(w1t[:, 0:C], rows_val[k], preferred_element_type=f32)
        cm_k = jnp.dot(w1t[:, C:2 * C], trows_val[k], preferred_element_type=f32)
        h2_k = jnp.maximum(rm_k + cm_k + b1o, 0.0).astype(bf16)
        onehot = (ri == ik).astype(f32)
        a_col = jnp.dot(af_val, onehot, preferred_element_type=f32)
        he_k = jnp.maximum(a_col + bf_val, 0.0).astype(bf16)
        s2_k = (jnp.dot(w2o_s, h2_k, preferred_element_type=f32)
                + jnp.dot(w2e_s, he_k, preferred_element_type=f32) + b2_s)
        out2[k, :, :] = jax.nn.sigmoid(s2_k) * act2
    ki3 = jax.lax.broadcasted_iota(jnp.int32, (TI, C, N), 0)
    ji3 = jax.lax.broadcasted_iota(jnp.int32, (TI, C, N), 2)
    dmask3 = ji3 == (ki3 + ib * TI)
    zero_b = jnp.zeros((), bf16)
    one_b = jnp.ones((), bf16)
    mx[...] = jnp.maximum(
        mx[...], jnp.max(jnp.where(dmask3, zero_b, trows_val), axis=0).astype(f32))
    mn[...] = jnp.minimum(
        mn[...], jnp.min(jnp.where(dmask3, one_b, trows_val), axis=0).astype(f32))

    @pl.when(ib == NSTEPS - 1)
    def _():
        red = jnp.concatenate([mx[...], mn[...]], axis=0)
        s1 = (mlp(red, 'red1') * g(4) + mlp(x1m[...], 'op1') * g(3)
              + mlp(x0m[...], 'exp1') * g(2))
        out1[...] = jax.nn.sigmoid(s1) * g(8)


def kernel(x0, x1, x2, params, action):
    f32 = jnp.float32
    x0m = jnp.transpose(x0, (1, 0))
    x1m = jnp.transpose(x1, (0, 2, 1)).reshape(C, N)
    x2m = jnp.transpose(x2, (0, 1, 3, 2)).reshape(N, C, N)
    x2mb, x2tmb = _transpose(x2m)

    a = jnp.asarray(action, jnp.int32)
    bfs = [((a >> (NBITS - 1 - k)) & 1).astype(f32) for k in range(NBITS)]
    act0 = (bfs[0] + bfs[1] > 0).astype(f32)
    act1 = (bfs[2] + bfs[3] + bfs[4] > 0).astype(f32)
    act2 = (bfs[5] + bfs[6] > 0).astype(f32)
    gvec = jnp.stack(bfs + [act0, act1, act2] + [jnp.zeros(())] * 6)
    gates = jnp.broadcast_to(gvec[:, None], (16, 128)).astype(f32)

    weights = []
    wspecs = []
    for name in _NAMES:
        for suff in ('_W1', '_b1', '_W2', '_b2'):
            wgt = params[name + suff]
            if wgt.ndim == 1:
                wgt = wgt.reshape(-1, 1)
            else:
                wgt = wgt.T
            weights.append(wgt)
            wspecs.append(pl.BlockSpec(wgt.shape, lambda ib: (0, 0)))

    out0m, out1m, out2m = pl.pallas_call(
        _body,
        grid=(NSTEPS,),
        in_specs=[
            pl.BlockSpec((16, 128), lambda ib: (0, 0)),
            pl.BlockSpec((C, 1), lambda ib: (0, 0)),
            pl.BlockSpec((C, N), lambda ib: (0, 0)),
            pl.BlockSpec((TI, C, N), lambda ib: (ib, 0, 0)),
            pl.BlockSpec((TI, C, N), lambda ib: (ib, 0, 0)),
        ] + wspecs,
        out_specs=[
            pl.BlockSpec((O, 1), lambda ib: (0, 0)),
            pl.BlockSpec((O, N), lambda ib: (0, 0)),
            pl.BlockSpec((TI, O, N), lambda ib: (ib, 0, 0)),
        ],
        out_shape=[
            jax.ShapeDtypeStruct((O, 1), f32),
            jax.ShapeDtypeStruct((O, N), f32),
            jax.ShapeDtypeStruct((N, O, N), f32),
        ],
        scratch_shapes=[
            pltpu.VMEM((H, N), f32),
            pltpu.VMEM((H, N), f32),
            pltpu.VMEM((C, N), f32),
            pltpu.VMEM((C, N), f32),
        ],
        compiler_params=pltpu.CompilerParams(
            dimension_semantics=("arbitrary",),
        ),
    )(gates, x0m, x1m, x2mb, x2tmb, *weights)

    out0 = jnp.transpose(out0m, (1, 0))
    out1 = jnp.transpose(out1m[None], (0, 2, 1))
    out2 = jnp.transpose(out2m[None], (0, 1, 3, 2))
    return out0, out1, out2
